# Initial kernel scaffold; baseline (speedup 1.0000x reference)
import jax
import jax.numpy as jnp
from jax.experimental import pallas as pl


def kernel(Corpus_, batch_inputs, entity_embeddings, relation_embed, edge_list, edge_type, edge_embed, edge_list_nhop, edge_type_nhop, a0, a2_0, a1, a2_1, W, a_out, a2_out):
    raise NotImplementedError("write your pallas kernel here")


# speedup vs baseline: 1.4182x; 1.4182x over previous
import functools

import jax
import jax.numpy as jnp
from jax import lax
from jax.experimental import pallas as pl
from jax.experimental.pallas import tpu as pltpu
from jax.experimental.pallas import tpu_sc as plsc

NC = 2
NS = 16
LANES = 16

NHALF = 5120
NACC = 5248
ROWS_PER_TILE = NACC // NS
C = 80
RW = 128
NRELP = 208


def _lrelu_weight(logit):
    return jnp.exp(jnp.where(logit >= 0.0, -logit, -0.2 * logit))


def _mesh():
    return plsc.VectorSubcoreMesh(
        core_axis_name="c", subcore_axis_name="s", num_cores=NC, num_subcores=NS
    )


_CPARAMS = pltpu.CompilerParams(needs_layout_passes=False)


def _zero_1d(ref, nwords):
    zv = jnp.zeros((LANES,), jnp.float32)

    def body(i, carry):
        ref[pl.ds(i * LANES, LANES)] = zv
        return carry

    lax.fori_loop(0, nwords // LANES, body, 0, unroll=False)


def _local_idx(vd, c, s):
    vloc = vd - c * NHALF
    inr = (vloc >= 0) & (vloc < NHALF)
    trash = NHALF + s * 8 + (vd & 7)
    return jnp.where(inr, vloc, trash)


def _l1_body(ds_h, sr_h, q0_h, q1_h, qcat_h, vcat_h, sd0_h, ss0_h, sd1_h,
             ss1_h, z_h, out_h, rs_h,
             sd0v, ss0v, sd1v, ss1v, dsb, srb, dlb, dpad, q0b, q1b, qrows,
             vrows, orow, w0b, w1b, rs0v, rs1v, acc, gsem):
    c = lax.axis_index("c")
    s = lax.axis_index("s")
    epw = ds_h.shape[0] // NS
    n_chunks = epw // C

    row0 = pl.multiple_of(s * ROWS_PER_TILE, 8)
    pltpu.sync_copy(z_h, acc.at[pl.ds(row0, ROWS_PER_TILE)])
    _zero_1d(rs0v, NACC)
    _zero_1d(rs1v, NACC)

    pltpu.sync_copy(sd0_h, sd0v)
    pltpu.sync_copy(ss0_h, ss0v)
    pltpu.sync_copy(sd1_h, sd1v)
    pltpu.sync_copy(ss1_h, ss1v)
    plsc.subcore_barrier()

    lane = lax.iota(jnp.int32, LANES)

    def chunk_body(k, carry):
        base = pl.multiple_of(s * epw + k * C, 8)
        pltpu.sync_copy(ds_h.at[pl.ds(base, C)], dsb)
        pltpu.sync_copy(sr_h.at[pl.ds(base, C)], srb)
        pltpu.sync_copy(q0_h.at[pl.ds(base, C)], q0b)
        pltpu.sync_copy(q1_h.at[pl.ds(base, C)], q1b)
        pltpu.sync_copy(qcat_h.at[pl.ds(base, C)], qrows)
        pltpu.async_copy(vcat_h.at[srb], vrows, gsem).wait()

        for g in range(C // LANES):
            sl = pl.ds(g * LANES, LANES)
            vd = dsb[sl]
            vs = srb[sl]
            vloc = _local_idx(vd, c, s)
            dlb[sl] = vloc
            dpad[sl] = vloc
            l0 = (plsc.load_gather(sd0v, [vd]) + plsc.load_gather(ss0v, [vs])
                  + q0b[sl])
            l1 = (plsc.load_gather(sd1v, [vd]) + plsc.load_gather(ss1v, [vs])
                  + q1b[sl])
            w0 = _lrelu_weight(l0)
            w1 = _lrelu_weight(l1)
            w0b[sl] = w0
            w1b[sl] = w1
            for l in range(LANES):
                m = lane == l
                plsc.addupdate_scatter(rs0v, [vloc], w0, mask=m)
                plsc.addupdate_scatter(rs1v, [vloc], w1, mask=m)

        def edge_body(e, carry2):
            own = dpad[pl.ds(e, LANES)][0] < NHALF

            @pl.when(own)
            def _():
                w0 = w0b[pl.ds(e, LANES)][0]
                w1 = w1b[pl.ds(e, LANES)][0]
                for cg in range(4):
                    sl = pl.ds(cg * 16, 16)
                    orow[e, sl] = w0 * (vrows[e, sl] + qrows[e, sl])
                for cg in range(4, 8):
                    sl = pl.ds(cg * 16, 16)
                    orow[e, sl] = w1 * (vrows[e, sl] + qrows[e, sl])

            return carry2

        lax.fori_loop(0, C, edge_body, 0, unroll=False)

        pltpu.sync_copy(orow, acc.at[dlb], add=True)
        return carry

    lax.fori_loop(0, n_chunks, chunk_body, 0, unroll=False)
    plsc.subcore_barrier()

    pltpu.sync_copy(acc.at[pl.ds(row0, ROWS_PER_TILE)],
                    out_h.at[c, pl.ds(row0, ROWS_PER_TILE)])
    pltpu.sync_copy(rs0v, rs_h.at[c, s, 0])
    pltpu.sync_copy(rs1v, rs_h.at[c, s, 1])


def _l1_call(ds, sr, q0, q1, qcat, vcat, sd0, ss0, sd1, ss1):
    z = jnp.zeros((ROWS_PER_TILE, RW), jnp.float32)
    n = vcat.shape[0]
    f = functools.partial(
        pl.kernel,
        out_type=(jax.ShapeDtypeStruct((NC, NACC, RW), jnp.float32),
                  jax.ShapeDtypeStruct((NC, NS, 2, NACC), jnp.float32)),
        mesh=_mesh(),
        compiler_params=_CPARAMS,
        scratch_types=[
            pltpu.VMEM((n,), jnp.float32),
            pltpu.VMEM((n,), jnp.float32),
            pltpu.VMEM((n,), jnp.float32),
            pltpu.VMEM((n,), jnp.float32),
            pltpu.VMEM((C,), jnp.int32),
            pltpu.VMEM((C,), jnp.int32),
            pltpu.VMEM((C,), jnp.int32),
            pltpu.VMEM((C + 16,), jnp.int32),
            pltpu.VMEM((C,), jnp.float32),
            pltpu.VMEM((C,), jnp.float32),
            pltpu.VMEM((C, RW), jnp.float32),
            pltpu.VMEM((C, RW), jnp.float32),
            pltpu.VMEM((C, RW), jnp.float32),
            pltpu.VMEM((C + 16,), jnp.float32),
            pltpu.VMEM((C + 16,), jnp.float32),
            pltpu.VMEM((NACC,), jnp.float32),
            pltpu.VMEM((NACC,), jnp.float32),
            pltpu.VMEM_SHARED((NACC, RW), jnp.float32),
            pltpu.SemaphoreType.DMA,
        ],
    )(_l1_body)
    return f(ds, sr, q0, q1, qcat, vcat, sd0, ss0, sd1, ss1, z)


def _l2_body(ds_h, sr_h, t0_h, t1_h, v_h, sd_h, ss_h, relp_h, qs_h,
             z_h, out_h, rs_h,
             sdv, ssv, relpv, qsv, dsb, srb, dlb, dpad, t0pad, t1pad, vrows,
             orow, wb, rsv, acc, gsem):
    c = lax.axis_index("c")
    s = lax.axis_index("s")
    epw = ds_h.shape[0] // NS
    n_chunks = epw // C

    row0 = pl.multiple_of(s * ROWS_PER_TILE, 8)
    pltpu.sync_copy(z_h, acc.at[pl.ds(row0, ROWS_PER_TILE)])
    _zero_1d(rsv, NACC)

    pltpu.sync_copy(sd_h, sdv)
    pltpu.sync_copy(ss_h, ssv)
    pltpu.sync_copy(relp_h, relpv)
    pltpu.sync_copy(qs_h, qsv)
    plsc.subcore_barrier()

    lane = lax.iota(jnp.int32, LANES)

    def chunk_body(k, carry):
        base = pl.multiple_of(s * epw + k * C, 8)
        pltpu.sync_copy(ds_h.at[pl.ds(base, C)], dsb)
        pltpu.sync_copy(sr_h.at[pl.ds(base, C)], srb)
        pltpu.sync_copy(t0_h.at[pl.ds(base, C)], t0pad.at[pl.ds(0, C)])
        pltpu.sync_copy(t1_h.at[pl.ds(base, C)], t1pad.at[pl.ds(0, C)])
        pltpu.async_copy(v_h.at[srb], vrows, gsem).wait()

        for g in range(C // LANES):
            sl = pl.ds(g * LANES, LANES)
            vd = dsb[sl]
            vs = srb[sl]
            vt0 = t0pad[sl]
            vt1 = t1pad[sl]
            vloc = _local_idx(vd, c, s)
            dlb[sl] = vloc
            dpad[sl] = vloc
            logit = (plsc.load_gather(sdv, [vd]) + plsc.load_gather(ssv, [vs])
                     + plsc.load_gather(qsv, [vt0])
                     + plsc.load_gather(qsv, [vt1]))
            w = _lrelu_weight(logit)
            wb[sl] = w
            for l in range(LANES):
                plsc.addupdate_scatter(rsv, [vloc], w, mask=lane == l)

        def edge_body(e, carry2):
            own = dpad[pl.ds(e, LANES)][0] < NHALF

            @pl.when(own)
            def _():
                w = wb[pl.ds(e, LANES)][0]
                t0e = t0pad[pl.ds(e, LANES)][0]
                t1e = t1pad[pl.ds(e, LANES)][0]
                for cg in range(8):
                    sl = pl.ds(cg * 16, 16)
                    orow[e, sl] = w * (vrows[e, sl] + relpv[t0e, sl]
                                       + relpv[t1e, sl])

            return carry2

        lax.fori_loop(0, C, edge_body, 0, unroll=False)
        pltpu.sync_copy(orow, acc.at[dlb], add=True)
        return carry

    lax.fori_loop(0, n_chunks, chunk_body, 0, unroll=False)
    plsc.subcore_barrier()

    pltpu.sync_copy(acc.at[pl.ds(row0, ROWS_PER_TILE)],
                    out_h.at[c, pl.ds(row0, ROWS_PER_TILE)])
    pltpu.sync_copy(rsv, rs_h.at[c, s])


def _l2_call(ds, sr, t0, t1, v, sd, ss, relp, qs):
    z = jnp.zeros((ROWS_PER_TILE, RW), jnp.float32)
    n = v.shape[0]
    f = functools.partial(
        pl.kernel,
        out_type=(jax.ShapeDtypeStruct((NC, NACC, RW), jnp.float32),
                  jax.ShapeDtypeStruct((NC, NS, NACC), jnp.float32)),
        mesh=_mesh(),
        compiler_params=_CPARAMS,
        scratch_types=[
            pltpu.VMEM((n,), jnp.float32),
            pltpu.VMEM((n,), jnp.float32),
            pltpu.VMEM((NRELP, RW), jnp.float32),
            pltpu.VMEM((NRELP,), jnp.float32),
            pltpu.VMEM((C,), jnp.int32),
            pltpu.VMEM((C,), jnp.int32),
            pltpu.VMEM((C,), jnp.int32),
            pltpu.VMEM((C + 16,), jnp.int32),
            pltpu.VMEM((C + 16,), jnp.int32),
            pltpu.VMEM((C + 16,), jnp.int32),
            pltpu.VMEM((C, RW), jnp.float32),
            pltpu.VMEM((C, RW), jnp.float32),
            pltpu.VMEM((C + 16,), jnp.float32),
            pltpu.VMEM((NACC,), jnp.float32),
            pltpu.VMEM_SHARED((NACC, RW), jnp.float32),
            pltpu.SemaphoreType.DMA,
        ],
    )(_l2_body)
    return f(ds, sr, t0, t1, v, sd, ss, relp, qs, z)


def kernel(Corpus_, batch_inputs, entity_embeddings, relation_embed,
           edge_list, edge_type, edge_embed, edge_list_nhop, edge_type_nhop,
           a0, a2_0, a1, a2_1, W, a_out, a2_out):
    x = entity_embeddings
    n, nfeat = x.shape
    nhid = a0.shape[0]

    edge_all = jnp.concatenate([edge_list, edge_list_nhop], axis=1)
    ds = edge_all[0]
    sr = edge_all[1]
    ee_nhop = (relation_embed[edge_type_nhop[:, 0]]
               + relation_embed[edge_type_nhop[:, 1]])
    ee1 = jnp.concatenate([edge_embed, ee_nhop], axis=0)

    def split(a):
        return a[:, :nfeat], a[:, nfeat:2 * nfeat], a[:, 2 * nfeat:]

    A0_0, A1_0, A2_0 = split(a0)
    A0_1, A1_1, A2_1 = split(a1)
    v0 = a2_0[0]
    v1 = a2_1[0]
    P0_0 = x @ A0_0.T
    P0_1 = x @ A0_1.T
    V_0 = x @ A1_0.T
    V_1 = x @ A1_1.T
    vcat = jnp.concatenate([V_0, V_1], axis=1)
    qcat = ee1 @ jnp.concatenate([A2_0.T, A2_1.T], axis=1)
    sd0 = P0_0 @ v0
    ss0 = V_0 @ v0
    sd1 = P0_1 @ v1
    ss1 = V_1 @ v1
    q0 = ee1 @ (A2_0.T @ v0)
    q1 = ee1 @ (A2_1.T @ v1)

    acc1, rs1h = _l1_call(ds, sr, q0, q1, qcat, vcat, sd0, ss0, sd1, ss1)
    S = jnp.concatenate([acc1[0, :NHALF], acc1[1, :n - NHALF]], axis=0)
    rsp = jnp.sum(rs1h, axis=1)
    rs = jnp.concatenate([rsp[0, :, :NHALF], rsp[1, :, :n - NHALF]], axis=1)
    S0 = S[:, :nhid]
    S1 = S[:, nhid:2 * nhid]
    rs0 = rs[0]
    rs1 = rs[1]
    h0 = (rs0[:, None] * P0_0 + S0) / jnp.where(
        rs0 == 0.0, 1e-12, rs0)[:, None]
    h1 = (rs1[:, None] * P0_1 + S1) / jnp.where(
        rs1 == 0.0, 1e-12, rs1)[:, None]
    x2 = jnp.concatenate([jax.nn.elu(h0), jax.nn.elu(h1)], axis=1)

    nh2 = a_out.shape[0]
    out_relation_1 = relation_embed @ W
    nrel = out_relation_1.shape[0]
    A0o = a_out[:, :nh2]
    A1o = a_out[:, nh2:2 * nh2]
    A2o = a_out[:, 2 * nh2:]
    vo = a2_out[0]
    P0o = x2 @ A0o.T
    Vo = x2 @ A1o.T
    sdo = P0o @ vo
    sso = Vo @ vo
    relp_small = out_relation_1 @ A2o.T
    relp = jnp.zeros((NRELP, nh2), jnp.float32).at[:nrel].set(relp_small)
    qs = jnp.zeros((NRELP,), jnp.float32).at[:nrel].set(relp_small @ vo)
    e1 = edge_type.shape[0]
    t0 = jnp.concatenate([edge_type, edge_type_nhop[:, 0]])
    t1 = jnp.concatenate([jnp.full((e1,), nrel, jnp.int32),
                          edge_type_nhop[:, 1]])

    acc2, rs2h = _l2_call(ds, sr, t0, t1, Vo, sdo, sso, relp, qs)
    So = jnp.concatenate([acc2[0, :NHALF], acc2[1, :n - NHALF]], axis=0)
    rsp2 = jnp.sum(rs2h, axis=1)
    rso = jnp.concatenate([rsp2[0, :NHALF], rsp2[1, :n - NHALF]], axis=0)
    ho = (rso[:, None] * P0o + So) / jnp.where(rso == 0.0, 1e-12, rso)[:, None]
    out = jax.nn.elu(ho)
    return (out, out_relation_1)


# speedup vs baseline: 1.7432x; 1.2292x over previous
import functools

import jax
import jax.numpy as jnp
from jax import lax
from jax.experimental import pallas as pl
from jax.experimental.pallas import tpu as pltpu
from jax.experimental.pallas import tpu_sc as plsc

NC = 2
NS = 16
LANES = 16

NHALF = 5120
NACC = 5248
ROWS_PER_TILE = NACC // NS
C = 48
RW = 128
NRELP = 208


def _lrelu_weight(logit):
    return jnp.exp(jnp.where(logit >= 0.0, -logit, -0.2 * logit))


def _mesh():
    return plsc.VectorSubcoreMesh(
        core_axis_name="c", subcore_axis_name="s", num_cores=NC, num_subcores=NS
    )


_CPARAMS = pltpu.CompilerParams(needs_layout_passes=False)


def _zero_1d(ref, nwords):
    zv = jnp.zeros((LANES,), jnp.float32)

    def body(i, carry):
        ref[pl.ds(i * LANES, LANES)] = zv
        return carry

    lax.fori_loop(0, nwords // LANES, body, 0, unroll=False)


def _local_idx(vd, c, s):
    vloc = vd - c * NHALF
    inr = (vloc >= 0) & (vloc < NHALF)
    trash = NHALF + s * 8 + (vd & 7)
    return jnp.where(inr, vloc, trash)


def _l1_body(ds_h, sr_h, q0_h, q1_h, qcat_h, vcat_h, sd0_h, ss0_h, sd1_h,
             ss1_h, z_h, out_h, rs_h,
             sd0v, ss0v, sd1v, ss1v,
             dsb0, srb0, q0b0, q1b0, qrows0, vrows0,
             dsb1, srb1, q0b1, q1b1, qrows1, vrows1,
             dlb, dpad, orow, w0b, w1b, rs0v, rs1v, acc,
             semA0, semA1, semG0, semG1):
    c = lax.axis_index("c")
    s = lax.axis_index("s")
    epw = ds_h.shape[0] // NS
    n_chunks = epw // C

    row0 = pl.multiple_of(s * ROWS_PER_TILE, 8)
    pltpu.sync_copy(z_h, acc.at[pl.ds(row0, ROWS_PER_TILE)])
    _zero_1d(rs0v, NACC)
    _zero_1d(rs1v, NACC)

    pltpu.sync_copy(sd0_h, sd0v)
    pltpu.sync_copy(ss0_h, ss0v)
    pltpu.sync_copy(sd1_h, sd1v)
    pltpu.sync_copy(ss1_h, ss1v)
    plsc.subcore_barrier()

    lane = lax.iota(jnp.int32, LANES)

    slots = ((dsb0, srb0, q0b0, q1b0, qrows0, vrows0, semA0, semG0),
             (dsb1, srb1, q0b1, q1b1, qrows1, vrows1, semA1, semG1))

    def _copiesA(k, sl):
        base = pl.multiple_of(s * epw + k * C, 8)
        return ((ds_h.at[pl.ds(base, C)], sl[0]),
                (sr_h.at[pl.ds(base, C)], sl[1]),
                (q0_h.at[pl.ds(base, C)], sl[2]),
                (q1_h.at[pl.ds(base, C)], sl[3]),
                (qcat_h.at[pl.ds(base, C)], sl[4]))

    def _issueA(k, sl):
        for src, dst in _copiesA(k, sl):
            pltpu.async_copy(src, dst, sl[6])

    def _waitA(k, sl):
        for src, dst in _copiesA(k, sl):
            pltpu.make_async_copy(src, dst, sl[6]).wait()

    _issueA(0, slots[0])

    def outer(g2, carry):
        for b in range(2):
            sl = slots[b]
            dsb, srb, q0b, q1b, qrows, vrows, semA, semG = sl
            k = g2 * 2 + b
            _waitA(k, sl)
            pltpu.async_copy(vcat_h.at[srb], vrows, semG)

            @pl.when(k + 1 < n_chunks)
            def _():
                _issueA(k + 1, slots[1 - b])

            for g in range(C // LANES):
                slc = pl.ds(g * LANES, LANES)
                vd = dsb[slc]
                vs = srb[slc]
                vloc = _local_idx(vd, c, s)
                dlb[slc] = vloc
                dpad[slc] = vloc
                l0 = (plsc.load_gather(sd0v, [vd])
                      + plsc.load_gather(ss0v, [vs]) + q0b[slc])
                l1 = (plsc.load_gather(sd1v, [vd])
                      + plsc.load_gather(ss1v, [vs]) + q1b[slc])
                w0 = _lrelu_weight(l0)
                w1 = _lrelu_weight(l1)
                w0b[slc] = w0
                w1b[slc] = w1
                for l in range(LANES):
                    m = lane == l
                    plsc.addupdate_scatter(rs0v, [vloc], w0, mask=m)
                    plsc.addupdate_scatter(rs1v, [vloc], w1, mask=m)

            pltpu.make_async_copy(vcat_h.at[srb], vrows, semG).wait()

            def edge_body(e, carry2):
                own = dpad[pl.ds(e, LANES)][0] < NHALF

                @pl.when(own)
                def _():
                    w0 = w0b[pl.ds(e, LANES)][0]
                    w1 = w1b[pl.ds(e, LANES)][0]
                    for cg in range(4):
                        slc = pl.ds(cg * 16, 16)
                        orow[e, slc] = w0 * (vrows[e, slc] + qrows[e, slc])
                    for cg in range(4, 8):
                        slc = pl.ds(cg * 16, 16)
                        orow[e, slc] = w1 * (vrows[e, slc] + qrows[e, slc])

                return carry2

            lax.fori_loop(0, C, edge_body, 0, unroll=False)

            pltpu.sync_copy(orow, acc.at[dlb], add=True)
        return carry

    lax.fori_loop(0, n_chunks // 2, outer, 0, unroll=False)
    plsc.subcore_barrier()

    pltpu.sync_copy(acc.at[pl.ds(row0, ROWS_PER_TILE)],
                    out_h.at[c, pl.ds(row0, ROWS_PER_TILE)])
    pltpu.sync_copy(rs0v, rs_h.at[c, s, 0])
    pltpu.sync_copy(rs1v, rs_h.at[c, s, 1])


def _l1_call(ds, sr, q0, q1, qcat, vcat, sd0, ss0, sd1, ss1):
    z = jnp.zeros((ROWS_PER_TILE, RW), jnp.float32)
    n = vcat.shape[0]
    f = functools.partial(
        pl.kernel,
        out_type=(jax.ShapeDtypeStruct((NC, NACC, RW), jnp.float32),
                  jax.ShapeDtypeStruct((NC, NS, 2, NACC), jnp.float32)),
        mesh=_mesh(),
        compiler_params=_CPARAMS,
        scratch_types=[
            pltpu.VMEM((n,), jnp.float32),
            pltpu.VMEM((n,), jnp.float32),
            pltpu.VMEM((n,), jnp.float32),
            pltpu.VMEM((n,), jnp.float32),
            pltpu.VMEM((C,), jnp.int32),
            pltpu.VMEM((C,), jnp.int32),
            pltpu.VMEM((C,), jnp.float32),
            pltpu.VMEM((C,), jnp.float32),
            pltpu.VMEM((C, RW), jnp.float32),
            pltpu.VMEM((C, RW), jnp.float32),
            pltpu.VMEM((C,), jnp.int32),
            pltpu.VMEM((C,), jnp.int32),
            pltpu.VMEM((C,), jnp.float32),
            pltpu.VMEM((C,), jnp.float32),
            pltpu.VMEM((C, RW), jnp.float32),
            pltpu.VMEM((C, RW), jnp.float32),
            pltpu.VMEM((C,), jnp.int32),
            pltpu.VMEM((C + 16,), jnp.int32),
            pltpu.VMEM((C, RW), jnp.float32),
            pltpu.VMEM((C + 16,), jnp.float32),
            pltpu.VMEM((C + 16,), jnp.float32),
            pltpu.VMEM((NACC,), jnp.float32),
            pltpu.VMEM((NACC,), jnp.float32),
            pltpu.VMEM_SHARED((NACC, RW), jnp.float32),
            pltpu.SemaphoreType.DMA,
            pltpu.SemaphoreType.DMA,
            pltpu.SemaphoreType.DMA,
            pltpu.SemaphoreType.DMA,
        ],
    )(_l1_body)
    return f(ds, sr, q0, q1, qcat, vcat, sd0, ss0, sd1, ss1, z)


def _l2_body(ds_h, sr_h, t0_h, t1_h, v_h, sd_h, ss_h, relp_h, qs_h,
             z_h, out_h, rs_h,
             sdv, ssv, relpv, qsv,
             dsb0, srb0, t0pad0, t1pad0, vrows0,
             dsb1, srb1, t0pad1, t1pad1, vrows1,
             dlb, dpad, orow, wb, rsv, acc,
             semA0, semA1, semG0, semG1):
    c = lax.axis_index("c")
    s = lax.axis_index("s")
    epw = ds_h.shape[0] // NS
    n_chunks = epw // C

    row0 = pl.multiple_of(s * ROWS_PER_TILE, 8)
    pltpu.sync_copy(z_h, acc.at[pl.ds(row0, ROWS_PER_TILE)])
    _zero_1d(rsv, NACC)

    pltpu.sync_copy(sd_h, sdv)
    pltpu.sync_copy(ss_h, ssv)
    pltpu.sync_copy(relp_h, relpv)
    pltpu.sync_copy(qs_h, qsv)
    plsc.subcore_barrier()

    lane = lax.iota(jnp.int32, LANES)

    slots = ((dsb0, srb0, t0pad0, t1pad0, vrows0, semA0, semG0),
             (dsb1, srb1, t0pad1, t1pad1, vrows1, semA1, semG1))

    def _copiesA(k, sl):
        base = pl.multiple_of(s * epw + k * C, 8)
        return ((ds_h.at[pl.ds(base, C)], sl[0]),
                (sr_h.at[pl.ds(base, C)], sl[1]),
                (t0_h.at[pl.ds(base, C)], sl[2].at[pl.ds(0, C)]),
                (t1_h.at[pl.ds(base, C)], sl[3].at[pl.ds(0, C)]))

    def _issueA(k, sl):
        for src, dst in _copiesA(k, sl):
            pltpu.async_copy(src, dst, sl[5])

    def _waitA(k, sl):
        for src, dst in _copiesA(k, sl):
            pltpu.make_async_copy(src, dst, sl[5]).wait()

    _issueA(0, slots[0])

    def outer(g2, carry):
        for b in range(2):
            sl = slots[b]
            dsb, srb, t0pad, t1pad, vrows, semA, semG = sl
            k = g2 * 2 + b
            _waitA(k, sl)
            pltpu.async_copy(v_h.at[srb], vrows, semG)

            @pl.when(k + 1 < n_chunks)
            def _():
                _issueA(k + 1, slots[1 - b])

            for g in range(C // LANES):
                slc = pl.ds(g * LANES, LANES)
                vd = dsb[slc]
                vs = srb[slc]
                vt0 = t0pad[slc]
                vt1 = t1pad[slc]
                vloc = _local_idx(vd, c, s)
                dlb[slc] = vloc
                dpad[slc] = vloc
                logit = (plsc.load_gather(sdv, [vd])
                         + plsc.load_gather(ssv, [vs])
                         + plsc.load_gather(qsv, [vt0])
                         + plsc.load_gather(qsv, [vt1]))
                w = _lrelu_weight(logit)
                wb[slc] = w
                for l in range(LANES):
                    plsc.addupdate_scatter(rsv, [vloc], w, mask=lane == l)

            pltpu.make_async_copy(v_h.at[srb], vrows, semG).wait()

            def edge_body(e, carry2):
                own = dpad[pl.ds(e, LANES)][0] < NHALF

                @pl.when(own)
                def _():
                    w = wb[pl.ds(e, LANES)][0]
                    t0e = t0pad[pl.ds(e, LANES)][0]
                    t1e = t1pad[pl.ds(e, LANES)][0]
                    for cg in range(8):
                        slc = pl.ds(cg * 16, 16)
                        orow[e, slc] = w * (vrows[e, slc] + relpv[t0e, slc]
                                            + relpv[t1e, slc])

                return carry2

            lax.fori_loop(0, C, edge_body, 0, unroll=False)
            pltpu.sync_copy(orow, acc.at[dlb], add=True)
        return carry

    lax.fori_loop(0, n_chunks // 2, outer, 0, unroll=False)
    plsc.subcore_barrier()

    pltpu.sync_copy(acc.at[pl.ds(row0, ROWS_PER_TILE)],
                    out_h.at[c, pl.ds(row0, ROWS_PER_TILE)])
    pltpu.sync_copy(rsv, rs_h.at[c, s])


def _l2_call(ds, sr, t0, t1, v, sd, ss, relp, qs):
    z = jnp.zeros((ROWS_PER_TILE, RW), jnp.float32)
    n = v.shape[0]
    f = functools.partial(
        pl.kernel,
        out_type=(jax.ShapeDtypeStruct((NC, NACC, RW), jnp.float32),
                  jax.ShapeDtypeStruct((NC, NS, NACC), jnp.float32)),
        mesh=_mesh(),
        compiler_params=_CPARAMS,
        scratch_types=[
            pltpu.VMEM((n,), jnp.float32),
            pltpu.VMEM((n,), jnp.float32),
            pltpu.VMEM((NRELP, RW), jnp.float32),
            pltpu.VMEM((NRELP,), jnp.float32),
            pltpu.VMEM((C,), jnp.int32),
            pltpu.VMEM((C,), jnp.int32),
            pltpu.VMEM((C + 16,), jnp.int32),
            pltpu.VMEM((C + 16,), jnp.int32),
            pltpu.VMEM((C, RW), jnp.float32),
            pltpu.VMEM((C,), jnp.int32),
            pltpu.VMEM((C,), jnp.int32),
            pltpu.VMEM((C + 16,), jnp.int32),
            pltpu.VMEM((C + 16,), jnp.int32),
            pltpu.VMEM((C, RW), jnp.float32),
            pltpu.VMEM((C,), jnp.int32),
            pltpu.VMEM((C + 16,), jnp.int32),
            pltpu.VMEM((C, RW), jnp.float32),
            pltpu.VMEM((C + 16,), jnp.float32),
            pltpu.VMEM((NACC,), jnp.float32),
            pltpu.VMEM_SHARED((NACC, RW), jnp.float32),
            pltpu.SemaphoreType.DMA,
            pltpu.SemaphoreType.DMA,
            pltpu.SemaphoreType.DMA,
            pltpu.SemaphoreType.DMA,
        ],
    )(_l2_body)
    return f(ds, sr, t0, t1, v, sd, ss, relp, qs, z)


def kernel(Corpus_, batch_inputs, entity_embeddings, relation_embed,
           edge_list, edge_type, edge_embed, edge_list_nhop, edge_type_nhop,
           a0, a2_0, a1, a2_1, W, a_out, a2_out):
    x = entity_embeddings
    n, nfeat = x.shape
    nhid = a0.shape[0]

    edge_all = jnp.concatenate([edge_list, edge_list_nhop], axis=1)
    ds = edge_all[0]
    sr = edge_all[1]
    ee_nhop = (relation_embed[edge_type_nhop[:, 0]]
               + relation_embed[edge_type_nhop[:, 1]])
    ee1 = jnp.concatenate([edge_embed, ee_nhop], axis=0)

    def split(a):
        return a[:, :nfeat], a[:, nfeat:2 * nfeat], a[:, 2 * nfeat:]

    A0_0, A1_0, A2_0 = split(a0)
    A0_1, A1_1, A2_1 = split(a1)
    v0 = a2_0[0]
    v1 = a2_1[0]
    P0_0 = x @ A0_0.T
    P0_1 = x @ A0_1.T
    V_0 = x @ A1_0.T
    V_1 = x @ A1_1.T
    vcat = jnp.concatenate([V_0, V_1], axis=1)
    qcat = ee1 @ jnp.concatenate([A2_0.T, A2_1.T], axis=1)
    sd0 = P0_0 @ v0
    ss0 = V_0 @ v0
    sd1 = P0_1 @ v1
    ss1 = V_1 @ v1
    q0 = ee1 @ (A2_0.T @ v0)
    q1 = ee1 @ (A2_1.T @ v1)

    acc1, rs1h = _l1_call(ds, sr, q0, q1, qcat, vcat, sd0, ss0, sd1, ss1)
    S = jnp.concatenate([acc1[0, :NHALF], acc1[1, :n - NHALF]], axis=0)
    rsp = jnp.sum(rs1h, axis=1)
    rs = jnp.concatenate([rsp[0, :, :NHALF], rsp[1, :, :n - NHALF]], axis=1)
    S0 = S[:, :nhid]
    S1 = S[:, nhid:2 * nhid]
    rs0 = rs[0]
    rs1 = rs[1]
    h0 = (rs0[:, None] * P0_0 + S0) / jnp.where(
        rs0 == 0.0, 1e-12, rs0)[:, None]
    h1 = (rs1[:, None] * P0_1 + S1) / jnp.where(
        rs1 == 0.0, 1e-12, rs1)[:, None]
    x2 = jnp.concatenate([jax.nn.elu(h0), jax.nn.elu(h1)], axis=1)

    nh2 = a_out.shape[0]
    out_relation_1 = relation_embed @ W
    nrel = out_relation_1.shape[0]
    A0o = a_out[:, :nh2]
    A1o = a_out[:, nh2:2 * nh2]
    A2o = a_out[:, 2 * nh2:]
    vo = a2_out[0]
    P0o = x2 @ A0o.T
    Vo = x2 @ A1o.T
    sdo = P0o @ vo
    sso = Vo @ vo
    relp_small = out_relation_1 @ A2o.T
    relp = jnp.zeros((NRELP, nh2), jnp.float32).at[:nrel].set(relp_small)
    qs = jnp.zeros((NRELP,), jnp.float32).at[:nrel].set(relp_small @ vo)
    e1 = edge_type.shape[0]
    t0 = jnp.concatenate([edge_type, edge_type_nhop[:, 0]])
    t1 = jnp.concatenate([jnp.full((e1,), nrel, jnp.int32),
                          edge_type_nhop[:, 1]])

    acc2, rs2h = _l2_call(ds, sr, t0, t1, Vo, sdo, sso, relp, qs)
    So = jnp.concatenate([acc2[0, :NHALF], acc2[1, :n - NHALF]], axis=0)
    rsp2 = jnp.sum(rs2h, axis=1)
    rso = jnp.concatenate([rsp2[0, :NHALF], rsp2[1, :n - NHALF]], axis=0)
    ho = (rso[:, None] * P0o + So) / jnp.where(rso == 0.0, 1e-12, rso)[:, None]
    out = jax.nn.elu(ho)
    return (out, out_relation_1)


# speedup vs baseline: 1.9763x; 1.1337x over previous
import functools

import jax
import jax.numpy as jnp
from jax import lax
from jax.experimental import pallas as pl
from jax.experimental.pallas import tpu as pltpu
from jax.experimental.pallas import tpu_sc as plsc

NC = 2
NS = 16
LANES = 16

NHALF = 5120
NACC = 5248
ROWS_PER_TILE = NACC // NS
C = 48
RW = 128
NRELP = 208


def _lrelu_weight(logit):
    return jnp.exp(jnp.where(logit >= 0.0, -logit, -0.2 * logit))


def _mesh():
    return plsc.VectorSubcoreMesh(
        core_axis_name="c", subcore_axis_name="s", num_cores=NC, num_subcores=NS
    )


_CPARAMS = pltpu.CompilerParams(needs_layout_passes=False)


def _zero_1d(ref, nwords):
    zv = jnp.zeros((LANES,), jnp.float32)

    def body(i, carry):
        ref[pl.ds(i * LANES, LANES)] = zv
        return carry

    lax.fori_loop(0, nwords // LANES, body, 0, unroll=False)


def _local_idx(vd, c, s):
    vloc = vd - c * NHALF
    inr = (vloc >= 0) & (vloc < NHALF)
    trash = NHALF + s * 8 + (vd & 7)
    return jnp.where(inr, vloc, trash)


def _l1_body(ds_h, sr_h, q0_h, q1_h, qcat_h, vcat_h, sd0_h, ss0_h, sd1_h,
             ss1_h, z_h, out_h, rs_h,
             sd0v, ss0v, sd1v, ss1v,
             dsb0, srb0, q0b0, q1b0, qrows0, vrows0, dlb0,
             dsb1, srb1, q0b1, q1b1, qrows1, vrows1, dlb1,
             dpad, w0b, w1b, rs0v, rs1v, acc,
             semA0, semA1, semG0, semG1, semS0, semS1):
    c = lax.axis_index("c")
    s = lax.axis_index("s")
    epw = ds_h.shape[0] // NS
    n_chunks = epw // C

    row0 = pl.multiple_of(s * ROWS_PER_TILE, 8)
    pltpu.sync_copy(z_h, acc.at[pl.ds(row0, ROWS_PER_TILE)])
    _zero_1d(rs0v, NACC)
    _zero_1d(rs1v, NACC)

    pltpu.sync_copy(sd0_h, sd0v)
    pltpu.sync_copy(ss0_h, ss0v)
    pltpu.sync_copy(sd1_h, sd1v)
    pltpu.sync_copy(ss1_h, ss1v)
    plsc.subcore_barrier()

    lane = lax.iota(jnp.int32, LANES)

    slots = ((dsb0, srb0, q0b0, q1b0, qrows0, vrows0, semA0, semG0, dlb0,
              semS0),
             (dsb1, srb1, q0b1, q1b1, qrows1, vrows1, semA1, semG1, dlb1,
              semS1))

    def _copiesA(k, sl):
        base = pl.multiple_of(s * epw + k * C, 8)
        return ((ds_h.at[pl.ds(base, C)], sl[0]),
                (sr_h.at[pl.ds(base, C)], sl[1]),
                (q0_h.at[pl.ds(base, C)], sl[2]),
                (q1_h.at[pl.ds(base, C)], sl[3]),
                (qcat_h.at[pl.ds(base, C)], sl[4]))

    def _issueA(k, sl):
        for src, dst in _copiesA(k, sl):
            pltpu.async_copy(src, dst, sl[6])

    def _waitA(k, sl):
        for src, dst in _copiesA(k, sl):
            pltpu.make_async_copy(src, dst, sl[6]).wait()

    _issueA(0, slots[0])

    def outer(g2, carry):
        for b in range(2):
            sl = slots[b]
            (dsb, srb, q0b, q1b, qrows, vrows, semA, semG, dlb, semS) = sl
            osl = slots[1 - b]
            k = g2 * 2 + b
            _waitA(k, sl)
            pltpu.async_copy(vcat_h.at[srb], vrows, semG)

            @pl.when(k >= 1)
            def _():
                pltpu.make_async_copy(osl[4], acc.at[osl[8]], osl[9]).wait()

            @pl.when(k + 1 < n_chunks)
            def _():
                _issueA(k + 1, osl)

            for g in range(C // LANES):
                slc = pl.ds(g * LANES, LANES)
                vd = dsb[slc]
                vs = srb[slc]
                vloc = _local_idx(vd, c, s)
                dlb[slc] = vloc
                dpad[slc] = vloc
                l0 = (plsc.load_gather(sd0v, [vd])
                      + plsc.load_gather(ss0v, [vs]) + q0b[slc])
                l1 = (plsc.load_gather(sd1v, [vd])
                      + plsc.load_gather(ss1v, [vs]) + q1b[slc])
                w0 = _lrelu_weight(l0)
                w1 = _lrelu_weight(l1)
                w0b[slc] = w0
                w1b[slc] = w1
                for l in range(LANES):
                    m = lane == l
                    plsc.addupdate_scatter(rs0v, [vloc], w0, mask=m)
                    plsc.addupdate_scatter(rs1v, [vloc], w1, mask=m)

            pltpu.make_async_copy(vcat_h.at[srb], vrows, semG).wait()

            def edge_body(e, carry2):
                own = dpad[pl.ds(e, LANES)][0] < NHALF

                @pl.when(own)
                def _():
                    w0 = w0b[pl.ds(e, LANES)][0]
                    w1 = w1b[pl.ds(e, LANES)][0]
                    for cg in range(4):
                        slc = pl.ds(cg * 16, 16)
                        qrows[e, slc] = w0 * (vrows[e, slc] + qrows[e, slc])
                    for cg in range(4, 8):
                        slc = pl.ds(cg * 16, 16)
                        qrows[e, slc] = w1 * (vrows[e, slc] + qrows[e, slc])

                return carry2

            lax.fori_loop(0, C, edge_body, 0, unroll=4)

            pltpu.async_copy(qrows, acc.at[dlb], semS, add=True)
        return carry

    lax.fori_loop(0, n_chunks // 2, outer, 0, unroll=False)
    pltpu.make_async_copy(slots[1][4], acc.at[slots[1][8]],
                          slots[1][9]).wait()
    plsc.subcore_barrier()

    pltpu.sync_copy(acc.at[pl.ds(row0, ROWS_PER_TILE)],
                    out_h.at[c, pl.ds(row0, ROWS_PER_TILE)])
    pltpu.sync_copy(rs0v, rs_h.at[c, s, 0])
    pltpu.sync_copy(rs1v, rs_h.at[c, s, 1])


def _l1_call(ds, sr, q0, q1, qcat, vcat, sd0, ss0, sd1, ss1):
    z = jnp.zeros((ROWS_PER_TILE, RW), jnp.float32)
    n = vcat.shape[0]
    f = functools.partial(
        pl.kernel,
        out_type=(jax.ShapeDtypeStruct((NC, NACC, RW), jnp.float32),
                  jax.ShapeDtypeStruct((NC, NS, 2, NACC), jnp.float32)),
        mesh=_mesh(),
        compiler_params=_CPARAMS,
        scratch_types=[
            pltpu.VMEM((n,), jnp.float32),
            pltpu.VMEM((n,), jnp.float32),
            pltpu.VMEM((n,), jnp.float32),
            pltpu.VMEM((n,), jnp.float32),
            pltpu.VMEM((C,), jnp.int32),
            pltpu.VMEM((C,), jnp.int32),
            pltpu.VMEM((C,), jnp.float32),
            pltpu.VMEM((C,), jnp.float32),
            pltpu.VMEM((C, RW), jnp.float32),
            pltpu.VMEM((C, RW), jnp.float32),
            pltpu.VMEM((C,), jnp.int32),
            pltpu.VMEM((C,), jnp.int32),
            pltpu.VMEM((C,), jnp.int32),
            pltpu.VMEM((C,), jnp.float32),
            pltpu.VMEM((C,), jnp.float32),
            pltpu.VMEM((C, RW), jnp.float32),
            pltpu.VMEM((C, RW), jnp.float32),
            pltpu.VMEM((C,), jnp.int32),
            pltpu.VMEM((C + 16,), jnp.int32),
            pltpu.VMEM((C + 16,), jnp.float32),
            pltpu.VMEM((C + 16,), jnp.float32),
            pltpu.VMEM((NACC,), jnp.float32),
            pltpu.VMEM((NACC,), jnp.float32),
            pltpu.VMEM_SHARED((NACC, RW), jnp.float32),
            pltpu.SemaphoreType.DMA,
            pltpu.SemaphoreType.DMA,
            pltpu.SemaphoreType.DMA,
            pltpu.SemaphoreType.DMA,
            pltpu.SemaphoreType.DMA,
            pltpu.SemaphoreType.DMA,
        ],
    )(_l1_body)
    return f(ds, sr, q0, q1, qcat, vcat, sd0, ss0, sd1, ss1, z)


def _l2_body(ds_h, sr_h, t0_h, t1_h, v_h, sd_h, ss_h, relp_h, qs_h,
             z_h, out_h, rs_h,
             sdv, ssv, relpv, qsv,
             dsb0, srb0, t0pad0, t1pad0, vrows0, dlb0,
             dsb1, srb1, t0pad1, t1pad1, vrows1, dlb1,
             dpad, wb, rsv, acc,
             semA0, semA1, semG0, semG1, semS0, semS1):
    c = lax.axis_index("c")
    s = lax.axis_index("s")
    epw = ds_h.shape[0] // NS
    n_chunks = epw // C

    row0 = pl.multiple_of(s * ROWS_PER_TILE, 8)
    pltpu.sync_copy(z_h, acc.at[pl.ds(row0, ROWS_PER_TILE)])
    _zero_1d(rsv, NACC)

    pltpu.sync_copy(sd_h, sdv)
    pltpu.sync_copy(ss_h, ssv)
    pltpu.sync_copy(relp_h, relpv)
    pltpu.sync_copy(qs_h, qsv)
    plsc.subcore_barrier()

    lane = lax.iota(jnp.int32, LANES)

    slots = ((dsb0, srb0, t0pad0, t1pad0, vrows0, semA0, semG0, dlb0,
              semS0),
             (dsb1, srb1, t0pad1, t1pad1, vrows1, semA1, semG1, dlb1,
              semS1))

    def _copiesA(k, sl):
        base = pl.multiple_of(s * epw + k * C, 8)
        return ((ds_h.at[pl.ds(base, C)], sl[0]),
                (sr_h.at[pl.ds(base, C)], sl[1]),
                (t0_h.at[pl.ds(base, C)], sl[2].at[pl.ds(0, C)]),
                (t1_h.at[pl.ds(base, C)], sl[3].at[pl.ds(0, C)]))

    def _issueA(k, sl):
        for src, dst in _copiesA(k, sl):
            pltpu.async_copy(src, dst, sl[5])

    def _waitA(k, sl):
        for src, dst in _copiesA(k, sl):
            pltpu.make_async_copy(src, dst, sl[5]).wait()

    _issueA(0, slots[0])

    def outer(g2, carry):
        for b in range(2):
            sl = slots[b]
            (dsb, srb, t0pad, t1pad, vrows, semA, semG, dlb, semS) = sl
            osl = slots[1 - b]
            k = g2 * 2 + b
            _waitA(k, sl)
            pltpu.async_copy(v_h.at[srb], vrows, semG)

            @pl.when(k >= 1)
            def _():
                pltpu.make_async_copy(osl[4], acc.at[osl[7]], osl[8]).wait()

            @pl.when(k + 1 < n_chunks)
            def _():
                _issueA(k + 1, osl)

            for g in range(C // LANES):
                slc = pl.ds(g * LANES, LANES)
                vd = dsb[slc]
                vs = srb[slc]
                vt0 = t0pad[slc]
                vt1 = t1pad[slc]
                vloc = _local_idx(vd, c, s)
                dlb[slc] = vloc
                dpad[slc] = vloc
                logit = (plsc.load_gather(sdv, [vd])
                         + plsc.load_gather(ssv, [vs])
                         + plsc.load_gather(qsv, [vt0])
                         + plsc.load_gather(qsv, [vt1]))
                w = _lrelu_weight(logit)
                wb[slc] = w
                for l in range(LANES):
                    plsc.addupdate_scatter(rsv, [vloc], w, mask=lane == l)

            pltpu.make_async_copy(v_h.at[srb], vrows, semG).wait()

            def edge_body(e, carry2):
                own = dpad[pl.ds(e, LANES)][0] < NHALF

                @pl.when(own)
                def _():
                    w = wb[pl.ds(e, LANES)][0]
                    t0e = t0pad[pl.ds(e, LANES)][0]
                    t1e = t1pad[pl.ds(e, LANES)][0]
                    for cg in range(8):
                        slc = pl.ds(cg * 16, 16)
                        vrows[e, slc] = w * (vrows[e, slc] + relpv[t0e, slc]
                                             + relpv[t1e, slc])

                return carry2

            lax.fori_loop(0, C, edge_body, 0, unroll=4)
            pltpu.async_copy(vrows, acc.at[dlb], semS, add=True)
        return carry

    lax.fori_loop(0, n_chunks // 2, outer, 0, unroll=False)
    pltpu.make_async_copy(slots[1][4], acc.at[slots[1][7]],
                          slots[1][8]).wait()
    plsc.subcore_barrier()

    pltpu.sync_copy(acc.at[pl.ds(row0, ROWS_PER_TILE)],
                    out_h.at[c, pl.ds(row0, ROWS_PER_TILE)])
    pltpu.sync_copy(rsv, rs_h.at[c, s])


def _l2_call(ds, sr, t0, t1, v, sd, ss, relp, qs):
    z = jnp.zeros((ROWS_PER_TILE, RW), jnp.float32)
    n = v.shape[0]
    f = functools.partial(
        pl.kernel,
        out_type=(jax.ShapeDtypeStruct((NC, NACC, RW), jnp.float32),
                  jax.ShapeDtypeStruct((NC, NS, NACC), jnp.float32)),
        mesh=_mesh(),
        compiler_params=_CPARAMS,
        scratch_types=[
            pltpu.VMEM((n,), jnp.float32),
            pltpu.VMEM((n,), jnp.float32),
            pltpu.VMEM((NRELP, RW), jnp.float32),
            pltpu.VMEM((NRELP,), jnp.float32),
            pltpu.VMEM((C,), jnp.int32),
            pltpu.VMEM((C,), jnp.int32),
            pltpu.VMEM((C + 16,), jnp.int32),
            pltpu.VMEM((C + 16,), jnp.int32),
            pltpu.VMEM((C, RW), jnp.float32),
            pltpu.VMEM((C,), jnp.int32),
            pltpu.VMEM((C,), jnp.int32),
            pltpu.VMEM((C,), jnp.int32),
            pltpu.VMEM((C + 16,), jnp.int32),
            pltpu.VMEM((C + 16,), jnp.int32),
            pltpu.VMEM((C, RW), jnp.float32),
            pltpu.VMEM((C,), jnp.int32),
            pltpu.VMEM((C + 16,), jnp.int32),
            pltpu.VMEM((C + 16,), jnp.float32),
            pltpu.VMEM((NACC,), jnp.float32),
            pltpu.VMEM_SHARED((NACC, RW), jnp.float32),
            pltpu.SemaphoreType.DMA,
            pltpu.SemaphoreType.DMA,
            pltpu.SemaphoreType.DMA,
            pltpu.SemaphoreType.DMA,
            pltpu.SemaphoreType.DMA,
            pltpu.SemaphoreType.DMA,
        ],
    )(_l2_body)
    return f(ds, sr, t0, t1, v, sd, ss, relp, qs, z)


def kernel(Corpus_, batch_inputs, entity_embeddings, relation_embed,
           edge_list, edge_type, edge_embed, edge_list_nhop, edge_type_nhop,
           a0, a2_0, a1, a2_1, W, a_out, a2_out):
    x = entity_embeddings
    n, nfeat = x.shape
    nhid = a0.shape[0]

    edge_all = jnp.concatenate([edge_list, edge_list_nhop], axis=1)
    ds = edge_all[0]
    sr = edge_all[1]
    ee_nhop = (relation_embed[edge_type_nhop[:, 0]]
               + relation_embed[edge_type_nhop[:, 1]])
    ee1 = jnp.concatenate([edge_embed, ee_nhop], axis=0)

    def split(a):
        return a[:, :nfeat], a[:, nfeat:2 * nfeat], a[:, 2 * nfeat:]

    A0_0, A1_0, A2_0 = split(a0)
    A0_1, A1_1, A2_1 = split(a1)
    v0 = a2_0[0]
    v1 = a2_1[0]
    P0_0 = x @ A0_0.T
    P0_1 = x @ A0_1.T
    V_0 = x @ A1_0.T
    V_1 = x @ A1_1.T
    vcat = jnp.concatenate([V_0, V_1], axis=1)
    qcat = ee1 @ jnp.concatenate([A2_0.T, A2_1.T], axis=1)
    sd0 = P0_0 @ v0
    ss0 = V_0 @ v0
    sd1 = P0_1 @ v1
    ss1 = V_1 @ v1
    q0 = ee1 @ (A2_0.T @ v0)
    q1 = ee1 @ (A2_1.T @ v1)

    acc1, rs1h = _l1_call(ds, sr, q0, q1, qcat, vcat, sd0, ss0, sd1, ss1)
    S = jnp.concatenate([acc1[0, :NHALF], acc1[1, :n - NHALF]], axis=0)
    rsp = jnp.sum(rs1h, axis=1)
    rs = jnp.concatenate([rsp[0, :, :NHALF], rsp[1, :, :n - NHALF]], axis=1)
    S0 = S[:, :nhid]
    S1 = S[:, nhid:2 * nhid]
    rs0 = rs[0]
    rs1 = rs[1]
    h0 = (rs0[:, None] * P0_0 + S0) / jnp.where(
        rs0 == 0.0, 1e-12, rs0)[:, None]
    h1 = (rs1[:, None] * P0_1 + S1) / jnp.where(
        rs1 == 0.0, 1e-12, rs1)[:, None]
    x2 = jnp.concatenate([jax.nn.elu(h0), jax.nn.elu(h1)], axis=1)

    nh2 = a_out.shape[0]
    out_relation_1 = relation_embed @ W
    nrel = out_relation_1.shape[0]
    A0o = a_out[:, :nh2]
    A1o = a_out[:, nh2:2 * nh2]
    A2o = a_out[:, 2 * nh2:]
    vo = a2_out[0]
    P0o = x2 @ A0o.T
    Vo = x2 @ A1o.T
    sdo = P0o @ vo
    sso = Vo @ vo
    relp_small = out_relation_1 @ A2o.T
    relp = jnp.zeros((NRELP, nh2), jnp.float32).at[:nrel].set(relp_small)
    qs = jnp.zeros((NRELP,), jnp.float32).at[:nrel].set(relp_small @ vo)
    e1 = edge_type.shape[0]
    t0 = jnp.concatenate([edge_type, edge_type_nhop[:, 0]])
    t1 = jnp.concatenate([jnp.full((e1,), nrel, jnp.int32),
                          edge_type_nhop[:, 1]])

    acc2, rs2h = _l2_call(ds, sr, t0, t1, Vo, sdo, sso, relp, qs)
    So = jnp.concatenate([acc2[0, :NHALF], acc2[1, :n - NHALF]], axis=0)
    rsp2 = jnp.sum(rs2h, axis=1)
    rso = jnp.concatenate([rsp2[0, :NHALF], rsp2[1, :n - NHALF]], axis=0)
    ho = (rso[:, None] * P0o + So) / jnp.where(rso == 0.0, 1e-12, rso)[:, None]
    out = jax.nn.elu(ho)
    return (out, out_relation_1)
